# Initial kernel scaffold; baseline (speedup 1.0000x reference)
import functools

import jax
import jax.numpy as jnp
from jax import lax
from jax.experimental import pallas as pl
from jax.experimental.pallas import tpu as pltpu
from jax.experimental.pallas import tpu_sc as plsc

B = 64
NC = 2
NS = 16
NW = NC * NS
CHUNK = 128
NBUF = 4
LANES = 16


def _sc_body(xT_hbm, idxx_hbm, idxy_hbm, vals_hbm, out_hbm,
             y_sh, idxx_v, idxy_v, vals_v,
             gb0, gb1, gb2, gb3,
             gs0, gs1, gs2, gs3, ss0, ss1, ss2, ss3):
  cpt = idxx_v.shape[0]
  n_out = y_sh.shape[0]
  cid = lax.axis_index("c")
  sid = lax.axis_index("s")
  wid = cid * NS + sid

  bufs = [gb0, gb1, gb2, gb3]
  gsems = [gs0, gs1, gs2, gs3]
  ssems = [ss0, ss1, ss2, ss3]

  pltpu.sync_copy(idxx_hbm.at[wid], idxx_v)
  pltpu.sync_copy(idxy_hbm.at[wid], idxy_v)
  pltpu.sync_copy(vals_hbm.at[wid], vals_v)

  zero16 = jnp.zeros((LANES,), jnp.float32)

  @pl.loop(0, CHUNK)
  def _zero_buf(i):
    for k in range(B // LANES):
      gb0[i, pl.ds(k * LANES, LANES)] = zero16

  rows_per = n_out // NS

  @pl.loop(0, rows_per // CHUNK)
  def _zero_sh(k):
    pltpu.sync_copy(gb0, y_sh.at[pl.ds(sid * rows_per + k * CHUNK, CHUNK)])

  plsc.subcore_barrier()

  def g_start(cc, p):
    pltpu.async_copy(xT_hbm.at[idxx_v.at[cc]], bufs[p], gsems[p])

  def g_wait(cc, p):
    pltpu.make_async_copy(xT_hbm.at[idxx_v.at[cc]], bufs[p], gsems[p]).wait()

  def s_start(cc, p):
    pltpu.async_copy(bufs[p], y_sh.at[idxy_v.at[cc]], ssems[p], add=True)

  def s_wait(cc, p):
    pltpu.make_async_copy(bufs[p], y_sh.at[idxy_v.at[cc]], ssems[p]).wait()

  for c in range(NBUF - 1):
    g_start(c, c)

  @pl.loop(0, cpt, step=NBUF)
  def _main(c4):
    for b in range(NBUF):
      cc = c4 + b
      gbuf = bufs[b]
      g_wait(cc, b)

      @pl.loop(0, CHUNK // LANES)
      def _scale(i16):
        vv = vals_v[cc, pl.ds(i16 * LANES, LANES)]
        for j in range(LANES):
          bc = jnp.take(vv, jnp.full((LANES,), j, jnp.int32),
                        mode="promise_in_bounds")
          it = i16 * LANES + j
          for k in range(B // LANES):
            sl = pl.ds(k * LANES, LANES)
            gbuf[it, sl] = gbuf[it, sl] * bc

      s_start(cc, b)

      @pl.when(cc >= 1)
      def _drain_prev_scatter():
        s_wait(cc - 1, (b - 1) % NBUF)

      @pl.when(cc + NBUF - 1 < cpt)
      def _refill():
        g_start(cc + NBUF - 1, (b + NBUF - 1) % NBUF)

  s_wait(cpt - 1, (cpt - 1) % NBUF)
  plsc.subcore_barrier()

  pltpu.sync_copy(y_sh.at[pl.ds(sid * rows_per, rows_per)],
                  out_hbm.at[cid].at[pl.ds(sid * rows_per, rows_per)])


def _make_sc_spmm(n_out, cpt):
  mesh = plsc.VectorSubcoreMesh(core_axis_name="c", subcore_axis_name="s")
  return pl.kernel(
      _sc_body,
      out_type=jax.ShapeDtypeStruct((NC, n_out, B), jnp.float32),
      mesh=mesh,
      scratch_types=[
          pltpu.VMEM_SHARED((n_out, B), jnp.float32),
          pltpu.VMEM((cpt, CHUNK), jnp.int32),
          pltpu.VMEM((cpt, CHUNK), jnp.int32),
          pltpu.VMEM((cpt, CHUNK), jnp.float32),
      ] + [pltpu.VMEM((CHUNK, B), jnp.float32) for _ in range(NBUF)]
        + [pltpu.SemaphoreType.DMA for _ in range(2 * NBUF)],
  )


def _transpose_in(x):
  n_in = x.shape[1]
  blk = 1024

  def body(x_ref, o_ref):
    o_ref[...] = x_ref[...].T

  return pl.pallas_call(
      body,
      grid=(n_in // blk,),
      in_specs=[pl.BlockSpec((B, blk), lambda j: (0, j))],
      out_specs=pl.BlockSpec((blk, B), lambda j: (j, 0)),
      out_shape=jax.ShapeDtypeStruct((n_in, B), jnp.float32),
  )(x)


def _combine_out(partials):
  n_out = partials.shape[1]
  blk = 1024

  def body(p_ref, o_ref):
    o_ref[...] = (p_ref[0] + p_ref[1]).T

  return pl.pallas_call(
      body,
      grid=(n_out // blk,),
      in_specs=[pl.BlockSpec((2, blk, B), lambda j: (0, j, 0))],
      out_specs=pl.BlockSpec((B, blk), lambda j: (0, j)),
      out_shape=jax.ShapeDtypeStruct((B, n_out), jnp.float32),
  )(partials)


@jax.jit
def kernel(x, vals, idx_xs, idx_ys):
  n_out = x.shape[1]
  nnz = vals.shape[0]
  per_transfer = NW * CHUNK
  cpt = -(-nnz // per_transfer)
  if cpt % NBUF:
    cpt += NBUF - cpt % NBUF
  items = cpt * per_transfer
  pad = items - nnz

  idxx = jnp.concatenate([idx_xs, jnp.zeros((pad,), jnp.int32)])
  idxy = jnp.concatenate([idx_ys, jnp.zeros((pad,), jnp.int32)])
  v = jnp.concatenate([vals, jnp.zeros((pad,), jnp.float32)])
  idxx = idxx.reshape(NW, cpt, CHUNK)
  idxy = idxy.reshape(NW, cpt, CHUNK)
  v = v.reshape(NW, cpt, CHUNK)

  xT = _transpose_in(x)
  partials = _make_sc_spmm(n_out, cpt)(xT, idxx, idxy, v)
  return _combine_out(partials)


# speedup vs baseline: 5.9885x; 5.9885x over previous
import functools

import jax
import jax.numpy as jnp
from jax import lax
from jax.experimental import pallas as pl
from jax.experimental.pallas import tpu as pltpu
from jax.experimental.pallas import tpu_sc as plsc

B = 64
NC = 2
NS = 16
NW = NC * NS
CHUNK = 128
NBUF = 4
LANES = 16


def _sc_body(xT_hbm, idxx_hbm, idxy_hbm, vals_hbm, out_hbm,
             y_sh, idxx_v, idxy_v, vals_v,
             gb0, gb1, gb2, gb3,
             gs0, gs1, gs2, gs3, ss0, ss1, ss2, ss3):
  cpt = idxx_v.shape[0]
  n_out = y_sh.shape[0]
  cid = lax.axis_index("c")
  sid = lax.axis_index("s")
  wid = cid * NS + sid

  bufs = [gb0, gb1, gb2, gb3]
  gsems = [gs0, gs1, gs2, gs3]
  ssems = [ss0, ss1, ss2, ss3]

  pltpu.sync_copy(idxx_hbm.at[wid], idxx_v)
  pltpu.sync_copy(idxy_hbm.at[wid], idxy_v)
  pltpu.sync_copy(vals_hbm.at[wid], vals_v)

  zero16 = jnp.zeros((LANES,), jnp.float32)

  @pl.loop(0, CHUNK)
  def _zero_buf(i):
    for k in range(B // LANES):
      gb0[i, pl.ds(k * LANES, LANES)] = zero16

  rows_per = n_out // NS

  @pl.loop(0, rows_per // CHUNK)
  def _zero_sh(k):
    pltpu.sync_copy(gb0, y_sh.at[pl.ds(sid * rows_per + k * CHUNK, CHUNK)])

  plsc.subcore_barrier()

  def g_start(cc, p):
    pltpu.async_copy(xT_hbm.at[idxx_v.at[cc]], bufs[p], gsems[p])

  def g_wait(cc, p):
    pltpu.make_async_copy(xT_hbm.at[idxx_v.at[cc]], bufs[p], gsems[p]).wait()

  def s_start(cc, p):
    pltpu.async_copy(bufs[p], y_sh.at[idxy_v.at[cc]], ssems[p], add=True)

  def s_wait(cc, p):
    pltpu.make_async_copy(bufs[p], y_sh.at[idxy_v.at[cc]], ssems[p]).wait()

  for c in range(NBUF - 1):
    g_start(c, c)

  @pl.loop(0, cpt, step=NBUF)
  def _main(c4):
    for b in range(NBUF):
      cc = c4 + b
      gbuf = bufs[b]
      g_wait(cc, b)

      @pl.loop(0, CHUNK // LANES)
      def _scale(i16):
        vv = vals_v[cc, pl.ds(i16 * LANES, LANES)]
        for j in range(LANES):
          bc = lax.gather(
              vv, jnp.full((LANES, 1), j, jnp.int32),
              lax.GatherDimensionNumbers(offset_dims=(),
                                         collapsed_slice_dims=(0,),
                                         start_index_map=(0,)),
              slice_sizes=(1,),
              mode=lax.GatherScatterMode.PROMISE_IN_BOUNDS)
          it = i16 * LANES + j
          for k in range(B // LANES):
            sl = pl.ds(k * LANES, LANES)
            gbuf[it, sl] = gbuf[it, sl] * bc

      s_start(cc, b)

      @pl.when(cc >= 1)
      def _drain_prev_scatter():
        s_wait(cc - 1, (b - 1) % NBUF)

      @pl.when(cc + NBUF - 1 < cpt)
      def _refill():
        g_start(cc + NBUF - 1, (b + NBUF - 1) % NBUF)

  s_wait(cpt - 1, (cpt - 1) % NBUF)
  plsc.subcore_barrier()

  pltpu.sync_copy(y_sh.at[pl.ds(sid * rows_per, rows_per)],
                  out_hbm.at[cid].at[pl.ds(sid * rows_per, rows_per)])


def _make_sc_spmm(n_out, cpt):
  mesh = plsc.VectorSubcoreMesh(core_axis_name="c", subcore_axis_name="s")
  return pl.kernel(
      _sc_body,
      out_type=jax.ShapeDtypeStruct((NC, n_out, B), jnp.float32),
      mesh=mesh,
      scratch_types=[
          pltpu.VMEM_SHARED((n_out, B), jnp.float32),
          pltpu.VMEM((cpt, CHUNK), jnp.int32),
          pltpu.VMEM((cpt, CHUNK), jnp.int32),
          pltpu.VMEM((cpt, CHUNK), jnp.float32),
      ] + [pltpu.VMEM((CHUNK, B), jnp.float32) for _ in range(NBUF)]
        + [pltpu.SemaphoreType.DMA for _ in range(2 * NBUF)],
      compiler_params=pltpu.CompilerParams(use_tc_tiling_on_sc=False),
  )


def _transpose_in(x):
  n_in = x.shape[1]
  blk = 1024

  def body(x_ref, o_ref):
    o_ref[...] = x_ref[...].T

  return pl.pallas_call(
      body,
      grid=(n_in // blk,),
      in_specs=[pl.BlockSpec((B, blk), lambda j: (0, j))],
      out_specs=pl.BlockSpec((blk, B), lambda j: (j, 0)),
      out_shape=jax.ShapeDtypeStruct((n_in, B), jnp.float32),
  )(x)


def _combine_out(partials):
  n_out = partials.shape[1]
  blk = 1024

  def body(p_ref, o_ref):
    o_ref[...] = (p_ref[0] + p_ref[1]).T

  return pl.pallas_call(
      body,
      grid=(n_out // blk,),
      in_specs=[pl.BlockSpec((2, blk, B), lambda j: (0, j, 0))],
      out_specs=pl.BlockSpec((B, blk), lambda j: (0, j)),
      out_shape=jax.ShapeDtypeStruct((B, n_out), jnp.float32),
  )(partials)


@jax.jit
def kernel(x, vals, idx_xs, idx_ys):
  n_out = x.shape[1]
  nnz = vals.shape[0]
  per_transfer = NW * CHUNK
  cpt = -(-nnz // per_transfer)
  if cpt % NBUF:
    cpt += NBUF - cpt % NBUF
  items = cpt * per_transfer
  pad = items - nnz

  idxx = jnp.concatenate([idx_xs, jnp.zeros((pad,), jnp.int32)])
  idxy = jnp.concatenate([idx_ys, jnp.zeros((pad,), jnp.int32)])
  v = jnp.concatenate([vals, jnp.zeros((pad,), jnp.float32)])
  idxx = idxx.reshape(NW, cpt, CHUNK)
  idxy = idxy.reshape(NW, cpt, CHUNK)
  v = v.reshape(NW, cpt, CHUNK)

  xT = _transpose_in(x)
  partials = _make_sc_spmm(n_out, cpt)(xT, idxx, idxy, v)
  return _combine_out(partials)


# speedup vs baseline: 8.4238x; 1.4067x over previous
import functools

import jax
import jax.numpy as jnp
from jax import lax
from jax.experimental import pallas as pl
from jax.experimental.pallas import tpu as pltpu
from jax.experimental.pallas import tpu_sc as plsc

B = 64
NC = 2
NS = 16
CPS = B // NS
NPAIR = CPS // 2
CHUNKI = 2048
LANES = 16
GUNROLL = 4


def _sc_body(xpack_hbm, idxx_hbm, idxy_hbm, vals_hbm, out_hbm,
             a0, a1, acc0, acc1, acc2, acc3,
             bx0, by0, bv0, bx1, by1, bv1, sem0, sem1):
  n = a0.shape[0]
  n_chunks = idxx_hbm.shape[1]
  cid = lax.axis_index("c")
  sid = lax.axis_index("s")

  accs = [acc0, acc1, acc2, acc3]
  pairs = [a0, a1]
  bufs = [(bx0, by0, bv0, sem0), (bx1, by1, bv1, sem1)]

  for k in range(NPAIR):
    pltpu.sync_copy(xpack_hbm.at[sid * NPAIR + k], pairs[k])

  zero16 = jnp.zeros((LANES,), jnp.float32)

  @pl.loop(0, n // LANES)
  def _zero(i):
    for acc in accs:
      acc[pl.ds(i * LANES, LANES)] = zero16

  def issue(chunk, b):
    bx, by, bv, sem = bufs[b]
    pltpu.async_copy(idxx_hbm.at[cid].at[chunk], bx, sem)
    pltpu.async_copy(idxy_hbm.at[cid].at[chunk], by, sem)
    pltpu.async_copy(vals_hbm.at[cid].at[chunk], bv, sem)

  def wait(chunk, b):
    bx, by, bv, sem = bufs[b]
    pltpu.make_async_copy(idxx_hbm.at[cid].at[chunk], bx, sem).wait()
    pltpu.make_async_copy(idxy_hbm.at[cid].at[chunk], by, sem).wait()
    pltpu.make_async_copy(vals_hbm.at[cid].at[chunk], bv, sem).wait()

  issue(0, 0)
  issue(1, 1)

  himask = jnp.full((LANES,), -65536, jnp.int32)

  @pl.loop(0, n_chunks, step=2)
  def _main(h):
    for b in range(2):
      cc = h + b
      bx, by, bv, _ = bufs[b]
      wait(cc, b)

      @pl.loop(0, CHUNKI // LANES, unroll=GUNROLL)
      def _group(g):
        sl = pl.ds(g * LANES, LANES)
        vx = bx[sl]
        vy = by[sl]
        vv = bv[sl]
        for k in range(NPAIR):
          gp = plsc.load_gather(pairs[k], [vx])
          gi = plsc.bitcast(gp, jnp.int32)
          xe = plsc.bitcast(gi << 16, jnp.float32)
          xo = plsc.bitcast(gi & himask, jnp.float32)
          plsc.addupdate_scatter(accs[2 * k], [vy], xe * vv)
          plsc.addupdate_scatter(accs[2 * k + 1], [vy], xo * vv)

      @pl.when(cc + 2 < n_chunks)
      def _refill():
        issue(cc + 2, b)

  for k in range(CPS):
    pltpu.sync_copy(accs[k], out_hbm.at[cid].at[sid].at[k])


def _make_sc_spmm(n, n_chunks):
  mesh = plsc.VectorSubcoreMesh(core_axis_name="c", subcore_axis_name="s")
  return pl.kernel(
      _sc_body,
      out_type=jax.ShapeDtypeStruct((NC, NS, CPS, n), jnp.float32),
      mesh=mesh,
      scratch_types=[pltpu.VMEM((n,), jnp.float32) for _ in range(2 + CPS)]
      + [
          pltpu.VMEM((CHUNKI,), jnp.int32),
          pltpu.VMEM((CHUNKI,), jnp.int32),
          pltpu.VMEM((CHUNKI,), jnp.float32),
          pltpu.VMEM((CHUNKI,), jnp.int32),
          pltpu.VMEM((CHUNKI,), jnp.int32),
          pltpu.VMEM((CHUNKI,), jnp.float32),
          pltpu.SemaphoreType.DMA,
          pltpu.SemaphoreType.DMA,
      ],
      compiler_params=pltpu.CompilerParams(
          use_tc_tiling_on_sc=False, needs_layout_passes=False
      ),
  )


def _combine_out(parts):
  n = parts.shape[-1]
  blk = 2048

  def body(p_ref, o_ref):
    p = p_ref[...]
    o_ref[...] = (p[0] + p[1]).reshape(B, blk)

  return pl.pallas_call(
      body,
      grid=(n // blk,),
      in_specs=[pl.BlockSpec((NC, NS, CPS, blk), lambda j: (0, 0, 0, j))],
      out_specs=pl.BlockSpec((B, blk), lambda j: (0, j)),
      out_shape=jax.ShapeDtypeStruct((B, n), jnp.float32),
  )(parts)


@jax.jit
def kernel(x, vals, idx_xs, idx_ys):
  n = x.shape[1]
  nnz = vals.shape[0]
  per_round = NC * CHUNKI
  n_chunks = -(-nnz // per_round)
  if n_chunks % 2:
    n_chunks += 1
  items = n_chunks * per_round
  pad = items - nnz

  idxx = jnp.concatenate([idx_xs, jnp.zeros((pad,), jnp.int32)])
  idxy = jnp.concatenate([idx_ys, jnp.zeros((pad,), jnp.int32)])
  v = jnp.concatenate([vals, jnp.zeros((pad,), jnp.float32)])
  idxx = idxx.reshape(NC, n_chunks, CHUNKI)
  idxy = idxy.reshape(NC, n_chunks, CHUNKI)
  v = v.reshape(NC, n_chunks, CHUNKI)

  xb = x.astype(jnp.bfloat16)
  xpack = lax.bitcast_convert_type(
      xb.reshape(B // 2, 2, n).transpose(0, 2, 1), jnp.float32
  )

  parts = _make_sc_spmm(n, n_chunks)(xpack, idxx, idxy, v)
  return _combine_out(parts)


# speedup vs baseline: 14.5086x; 1.7223x over previous
import functools

import jax
import jax.numpy as jnp
from jax import lax
from jax.experimental import pallas as pl
from jax.experimental.pallas import tpu as pltpu
from jax.experimental.pallas import tpu_sc as plsc

B = 64
NC = 2
NS = 16
CPS = B // NS
NPAIR = CPS // 2
CHUNKI = 2048
LANES = 16
GUNROLL = 4


def _sc_body(xpack_hbm, idxx_hbm, idxy_hbm, vals_hbm, out_hbm,
             a0, a1, acc0, acc1, acc2, acc3,
             bx0, by0, bv0, bx1, by1, bv1, sem0, sem1):
  n = a0.shape[0]
  n_chunks = idxx_hbm.shape[1]
  cid = lax.axis_index("c")
  sid = lax.axis_index("s")

  accs = [acc0, acc1, acc2, acc3]
  pairs = [a0, a1]
  bufs = [(bx0, by0, bv0, sem0), (bx1, by1, bv1, sem1)]

  for k in range(NPAIR):
    pltpu.sync_copy(xpack_hbm.at[sid * NPAIR + k], pairs[k])

  zero16 = jnp.zeros((LANES,), jnp.float32)

  @plsc.parallel_loop(0, n // LANES)
  def _zero(i):
    for acc in accs:
      acc[pl.ds(i * LANES, LANES)] = zero16

  def issue(chunk, b):
    bx, by, bv, sem = bufs[b]
    pltpu.async_copy(idxx_hbm.at[cid].at[chunk], bx, sem)
    pltpu.async_copy(idxy_hbm.at[cid].at[chunk], by, sem)
    pltpu.async_copy(vals_hbm.at[cid].at[chunk], bv, sem)

  def wait(chunk, b):
    bx, by, bv, sem = bufs[b]
    pltpu.make_async_copy(idxx_hbm.at[cid].at[chunk], bx, sem).wait()
    pltpu.make_async_copy(idxy_hbm.at[cid].at[chunk], by, sem).wait()
    pltpu.make_async_copy(vals_hbm.at[cid].at[chunk], bv, sem).wait()

  issue(0, 0)
  issue(1, 1)

  himask = jnp.full((LANES,), -65536, jnp.int32)

  @pl.loop(0, n_chunks, step=2)
  def _main(h):
    for b in range(2):
      cc = h + b
      bx, by, bv, _ = bufs[b]
      wait(cc, b)

      @plsc.parallel_loop(0, CHUNKI // LANES, unroll=GUNROLL)
      def _group(g):
        sl = pl.ds(g * LANES, LANES)
        vx = bx[sl]
        vy = by[sl]
        vv = bv[sl]
        for k in range(NPAIR):
          gp = plsc.load_gather(pairs[k], [vx])
          gi = plsc.bitcast(gp, jnp.int32)
          xe = plsc.bitcast(gi << 16, jnp.float32)
          xo = plsc.bitcast(gi & himask, jnp.float32)
          plsc.addupdate_scatter(accs[2 * k], [vy], xe * vv)
          plsc.addupdate_scatter(accs[2 * k + 1], [vy], xo * vv)

      @pl.when(cc + 2 < n_chunks)
      def _refill():
        issue(cc + 2, b)

  for k in range(CPS):
    pltpu.sync_copy(accs[k], out_hbm.at[cid].at[sid].at[k])


def _make_sc_spmm(n, n_chunks):
  mesh = plsc.VectorSubcoreMesh(core_axis_name="c", subcore_axis_name="s")
  return pl.kernel(
      _sc_body,
      out_type=jax.ShapeDtypeStruct((NC, NS, CPS, n), jnp.float32),
      mesh=mesh,
      scratch_types=[pltpu.VMEM((n,), jnp.float32) for _ in range(2 + CPS)]
      + [
          pltpu.VMEM((CHUNKI,), jnp.int32),
          pltpu.VMEM((CHUNKI,), jnp.int32),
          pltpu.VMEM((CHUNKI,), jnp.float32),
          pltpu.VMEM((CHUNKI,), jnp.int32),
          pltpu.VMEM((CHUNKI,), jnp.int32),
          pltpu.VMEM((CHUNKI,), jnp.float32),
          pltpu.SemaphoreType.DMA,
          pltpu.SemaphoreType.DMA,
      ],
      compiler_params=pltpu.CompilerParams(
          use_tc_tiling_on_sc=False, needs_layout_passes=False
      ),
  )


def _combine_out(parts):
  n = parts.shape[-1]
  blk = 2048

  def body(p_ref, o_ref):
    p = p_ref[...]
    o_ref[...] = (p[0] + p[1]).reshape(B, blk)

  return pl.pallas_call(
      body,
      grid=(n // blk,),
      in_specs=[pl.BlockSpec((NC, NS, CPS, blk), lambda j: (0, 0, 0, j))],
      out_specs=pl.BlockSpec((B, blk), lambda j: (0, j)),
      out_shape=jax.ShapeDtypeStruct((B, n), jnp.float32),
  )(parts)


@jax.jit
def kernel(x, vals, idx_xs, idx_ys):
  n = x.shape[1]
  nnz = vals.shape[0]
  per_round = NC * CHUNKI
  n_chunks = -(-nnz // per_round)
  if n_chunks % 2:
    n_chunks += 1
  items = n_chunks * per_round
  pad = items - nnz

  idxx = jnp.concatenate([idx_xs, jnp.zeros((pad,), jnp.int32)])
  idxy = jnp.concatenate([idx_ys, jnp.zeros((pad,), jnp.int32)])
  v = jnp.concatenate([vals, jnp.zeros((pad,), jnp.float32)])
  idxx = idxx.reshape(NC, n_chunks, CHUNKI)
  idxy = idxy.reshape(NC, n_chunks, CHUNKI)
  v = v.reshape(NC, n_chunks, CHUNKI)

  xb = x.astype(jnp.bfloat16)
  xpack = lax.bitcast_convert_type(
      xb.reshape(B // 2, 2, n).transpose(0, 2, 1), jnp.float32
  )

  parts = _make_sc_spmm(n, n_chunks)(xpack, idxx, idxy, v)
  return _combine_out(parts)


# speedup vs baseline: 15.2456x; 1.0508x over previous
import functools

import jax
import jax.numpy as jnp
from jax import lax
from jax.experimental import pallas as pl
from jax.experimental.pallas import tpu as pltpu
from jax.experimental.pallas import tpu_sc as plsc

B = 64
NC = 2
NS = 16
CPS = B // NS
NPAIR = CPS // 2
CHUNKI = 2048
LANES = 16
GUNROLL = 8
XSHIFT = 14


def _sc_body(xpack_hbm, idxp_hbm, vals_hbm, out_hbm,
             a0, a1, acc0, acc1, acc2, acc3,
             bi0, bv0, bi1, bv1, sem0, sem1):
  n = a0.shape[0]
  n_chunks = idxp_hbm.shape[1]
  cid = lax.axis_index("c")
  sid = lax.axis_index("s")

  accs = [acc0, acc1, acc2, acc3]
  pairs = [a0, a1]
  bufs = [(bi0, bv0, sem0), (bi1, bv1, sem1)]

  for k in range(NPAIR):
    pltpu.sync_copy(xpack_hbm.at[sid * NPAIR + k], pairs[k])

  zero16 = jnp.zeros((LANES,), jnp.float32)

  @plsc.parallel_loop(0, n // LANES)
  def _zero(i):
    for acc in accs:
      acc[pl.ds(i * LANES, LANES)] = zero16

  def issue(chunk, b):
    bi, bv, sem = bufs[b]
    pltpu.async_copy(idxp_hbm.at[cid].at[chunk], bi, sem)
    pltpu.async_copy(vals_hbm.at[cid].at[chunk], bv, sem)

  def wait(chunk, b):
    bi, bv, sem = bufs[b]
    pltpu.make_async_copy(idxp_hbm.at[cid].at[chunk], bi, sem).wait()
    pltpu.make_async_copy(vals_hbm.at[cid].at[chunk], bv, sem).wait()

  issue(0, 0)
  issue(1, 1)

  himask = jnp.full((LANES,), -65536, jnp.int32)
  xmask = jnp.full((LANES,), (1 << XSHIFT) - 1, jnp.int32)

  @pl.loop(0, n_chunks, step=2)
  def _main(h):
    for b in range(2):
      cc = h + b
      bi, bv, _ = bufs[b]
      wait(cc, b)

      @plsc.parallel_loop(0, CHUNKI // LANES, unroll=GUNROLL)
      def _group(g):
        sl = pl.ds(g * LANES, LANES)
        vp = bi[sl]
        vv = bv[sl]
        vx = vp & xmask
        vy = lax.shift_right_logical(vp, XSHIFT)
        for k in range(NPAIR):
          gp = plsc.load_gather(pairs[k], [vx])
          gi = plsc.bitcast(gp, jnp.int32)
          xe = plsc.bitcast(gi << 16, jnp.float32)
          xo = plsc.bitcast(gi & himask, jnp.float32)
          plsc.addupdate_scatter(accs[2 * k], [vy], xe * vv)
          plsc.addupdate_scatter(accs[2 * k + 1], [vy], xo * vv)

      @pl.when(cc + 2 < n_chunks)
      def _refill():
        issue(cc + 2, b)

  for k in range(CPS):
    pltpu.sync_copy(accs[k], out_hbm.at[cid].at[sid * CPS + k])


def _make_sc_spmm(n, n_chunks):
  mesh = plsc.VectorSubcoreMesh(core_axis_name="c", subcore_axis_name="s")
  return pl.kernel(
      _sc_body,
      out_type=jax.ShapeDtypeStruct((NC, B, n), jnp.float32),
      mesh=mesh,
      scratch_types=[pltpu.VMEM((n,), jnp.float32) for _ in range(2 + CPS)]
      + [
          pltpu.VMEM((CHUNKI,), jnp.int32),
          pltpu.VMEM((CHUNKI,), jnp.float32),
          pltpu.VMEM((CHUNKI,), jnp.int32),
          pltpu.VMEM((CHUNKI,), jnp.float32),
          pltpu.SemaphoreType.DMA,
          pltpu.SemaphoreType.DMA,
      ],
      compiler_params=pltpu.CompilerParams(
          use_tc_tiling_on_sc=False, needs_layout_passes=False
      ),
  )


def _combine_out(parts):
  n = parts.shape[-1]
  blk = 2048

  def body(p_ref, o_ref):
    o_ref[...] = p_ref[0] + p_ref[1]

  return pl.pallas_call(
      body,
      grid=(n // blk,),
      in_specs=[pl.BlockSpec((NC, B, blk), lambda j: (0, 0, j))],
      out_specs=pl.BlockSpec((B, blk), lambda j: (0, j)),
      out_shape=jax.ShapeDtypeStruct((B, n), jnp.float32),
  )(parts)


@jax.jit
def kernel(x, vals, idx_xs, idx_ys):
  n = x.shape[1]
  nnz = vals.shape[0]
  per_round = NC * CHUNKI
  n_chunks = -(-nnz // per_round)
  if n_chunks % 2:
    n_chunks += 1
  items = n_chunks * per_round
  pad = items - nnz

  idxp = (idx_ys << XSHIFT) | idx_xs
  idxp = jnp.concatenate([idxp, jnp.zeros((pad,), jnp.int32)])
  v = jnp.concatenate([vals, jnp.zeros((pad,), jnp.float32)])
  idxp = idxp.reshape(NC, n_chunks, CHUNKI)
  v = v.reshape(NC, n_chunks, CHUNKI)

  xb = x.astype(jnp.bfloat16)
  xpack = lax.bitcast_convert_type(
      xb.reshape(B // 2, 2, n).transpose(0, 2, 1), jnp.float32
  )

  parts = _make_sc_spmm(n, n_chunks)(xpack, idxp, v)
  return _combine_out(parts)


# speedup vs baseline: 16.2793x; 1.0678x over previous
import functools

import jax
import jax.numpy as jnp
from jax import lax
from jax.experimental import pallas as pl
from jax.experimental.pallas import tpu as pltpu
from jax.experimental.pallas import tpu_sc as plsc

B = 64
NC = 2
NS = 16
CPS = B // NS
NPAIR = CPS // 2
CHUNKI = 2048
LANES = 16
GUNROLL = 8
XSHIFT = 14


def _sc_body(xpack_hbm, idxp_hbm, vals_hbm, out_hbm,
             a0, a1, acc0, acc1, acc2, acc3,
             bi0, bv0, bi1, bv1, sem0, sem1):
  n = a0.shape[0]
  n_chunks = idxp_hbm.shape[1]
  cid = lax.axis_index("c")
  sid = lax.axis_index("s")

  accs = [acc0, acc1, acc2, acc3]
  pairs = [a0, a1]
  bufs = [(bi0, bv0, sem0), (bi1, bv1, sem1)]

  for k in range(NPAIR):
    pltpu.sync_copy(xpack_hbm.at[sid * NPAIR + k], pairs[k])

  zero16 = jnp.zeros((LANES,), jnp.float32)

  @plsc.parallel_loop(0, n // LANES)
  def _zero(i):
    for acc in accs:
      acc[pl.ds(i * LANES, LANES)] = zero16

  def issue(chunk, b):
    bi, bv, sem = bufs[b]
    pltpu.async_copy(idxp_hbm.at[cid].at[chunk], bi, sem)
    pltpu.async_copy(vals_hbm.at[cid].at[chunk], bv, sem)

  def wait(chunk, b):
    bi, bv, sem = bufs[b]
    pltpu.make_async_copy(idxp_hbm.at[cid].at[chunk], bi, sem).wait()
    pltpu.make_async_copy(vals_hbm.at[cid].at[chunk], bv, sem).wait()

  issue(0, 0)
  issue(1, 1)

  himask = jnp.full((LANES,), -65536, jnp.int32)
  xmask = jnp.full((LANES,), (1 << XSHIFT) - 1, jnp.int32)

  @pl.loop(0, n_chunks, step=2)
  def _main(h):
    for b in range(2):
      cc = h + b
      bi, bv, _ = bufs[b]
      wait(cc, b)

      @plsc.parallel_loop(0, CHUNKI // LANES, unroll=GUNROLL)
      def _group(g):
        sl = pl.ds(g * LANES, LANES)
        vp = bi[sl]
        vv = bv[sl]
        vx = vp & xmask
        vy = lax.shift_right_logical(vp, XSHIFT)
        for k in range(NPAIR):
          gp = plsc.load_gather(pairs[k], [vx])
          gi = plsc.bitcast(gp, jnp.int32)
          xe = plsc.bitcast(gi << 16, jnp.float32)
          xo = plsc.bitcast(gi & himask, jnp.float32)
          plsc.addupdate_scatter(accs[2 * k], [vy], xe * vv)
          plsc.addupdate_scatter(accs[2 * k + 1], [vy], xo * vv)

      @pl.when(cc + 2 < n_chunks)
      def _refill():
        issue(cc + 2, b)

  for k in range(CPS):
    pltpu.sync_copy(accs[k], out_hbm.at[cid].at[sid * CPS + k])


def _make_sc_spmm(n, n_chunks):
  mesh = plsc.VectorSubcoreMesh(core_axis_name="c", subcore_axis_name="s")
  return pl.kernel(
      _sc_body,
      out_type=jax.ShapeDtypeStruct((NC, B, n), jnp.float32),
      mesh=mesh,
      scratch_types=[pltpu.VMEM((n,), jnp.float32) for _ in range(2 + CPS)]
      + [
          pltpu.VMEM((CHUNKI,), jnp.int32),
          pltpu.VMEM((CHUNKI,), jnp.float32),
          pltpu.VMEM((CHUNKI,), jnp.int32),
          pltpu.VMEM((CHUNKI,), jnp.float32),
          pltpu.SemaphoreType.DMA,
          pltpu.SemaphoreType.DMA,
      ],
      compiler_params=pltpu.CompilerParams(
          use_tc_tiling_on_sc=True, needs_layout_passes=False
      ),
  )


def _combine_out(parts):
  n = parts.shape[-1]
  blk = 2048

  def body(p_ref, o_ref):
    o_ref[...] = p_ref[0] + p_ref[1]

  return pl.pallas_call(
      body,
      grid=(n // blk,),
      in_specs=[pl.BlockSpec((NC, B, blk), lambda j: (0, 0, j))],
      out_specs=pl.BlockSpec((B, blk), lambda j: (0, j)),
      out_shape=jax.ShapeDtypeStruct((B, n), jnp.float32),
  )(parts)


@jax.jit
def kernel(x, vals, idx_xs, idx_ys):
  n = x.shape[1]
  nnz = vals.shape[0]
  per_round = NC * CHUNKI
  n_chunks = -(-nnz // per_round)
  if n_chunks % 2:
    n_chunks += 1
  items = n_chunks * per_round
  pad = items - nnz

  idxp = (idx_ys << XSHIFT) | idx_xs
  idxp = jnp.concatenate([idxp, jnp.zeros((pad,), jnp.int32)])
  v = jnp.concatenate([vals, jnp.zeros((pad,), jnp.float32)])
  idxp = idxp.reshape(NC, n_chunks, CHUNKI)
  v = v.reshape(NC, n_chunks, CHUNKI)

  xb = x.astype(jnp.bfloat16)
  xpack = lax.bitcast_convert_type(
      xb.reshape(B // 2, 2, n).transpose(0, 2, 1), jnp.float32
  )

  parts = _make_sc_spmm(n, n_chunks)(xpack, idxp, v)
  return _combine_out(parts)


# speedup vs baseline: 16.7209x; 1.0271x over previous
import functools

import jax
import jax.numpy as jnp
from jax import lax
from jax.experimental import pallas as pl
from jax.experimental.pallas import tpu as pltpu
from jax.experimental.pallas import tpu_sc as plsc

B = 64
NC = 2
NS = 16
CPS = B // NS
NPAIR = CPS // 2
CHUNKI = 2048
LANES = 16
GUNROLL = 8
XSHIFT = 14


def _sc_body(x_hbm, idxp_hbm, vals_hbm, out_hbm,
             a0, a1, acc0, acc1, acc2, acc3,
             bi0, bv0, bi1, bv1, t0, t1, sem0, sem1):
  n = a0.shape[0]
  n_chunks = idxp_hbm.shape[1]
  seg = t0.shape[0]
  cid = lax.axis_index("c")
  sid = lax.axis_index("s")

  accs = [acc0, acc1, acc2, acc3]
  pairs = [a0, a1]
  bufs = [(bi0, bv0, sem0), (bi1, bv1, sem1)]

  for k in range(NPAIR):
    r0 = sid * CPS + 2 * k
    for j in range(n // seg):
      pltpu.sync_copy(x_hbm.at[r0].at[pl.ds(j * seg, seg)], t0)
      pltpu.sync_copy(x_hbm.at[r0 + 1].at[pl.ds(j * seg, seg)], t1)

      @plsc.parallel_loop(0, seg // LANES, unroll=4)
      def _pack(g):
        sl = pl.ds(g * LANES, LANES)
        pw = plsc.bitcast(
            plsc.pack(t0[sl], t1[sl], format=plsc.PackFormat.INTERLEAVED),
            jnp.float32,
        )
        pairs[k][pl.ds(j * seg + g * LANES, LANES)] = pw

  zero16 = jnp.zeros((LANES,), jnp.float32)

  @plsc.parallel_loop(0, n // LANES)
  def _zero(i):
    for acc in accs:
      acc[pl.ds(i * LANES, LANES)] = zero16

  def issue(chunk, b):
    bi, bv, sem = bufs[b]
    pltpu.async_copy(idxp_hbm.at[cid].at[chunk], bi, sem)
    pltpu.async_copy(vals_hbm.at[cid].at[chunk], bv, sem)

  def wait(chunk, b):
    bi, bv, sem = bufs[b]
    pltpu.make_async_copy(idxp_hbm.at[cid].at[chunk], bi, sem).wait()
    pltpu.make_async_copy(vals_hbm.at[cid].at[chunk], bv, sem).wait()

  issue(0, 0)
  issue(1, 1)

  himask = jnp.full((LANES,), -65536, jnp.int32)
  xmask = jnp.full((LANES,), (1 << XSHIFT) - 1, jnp.int32)

  @pl.loop(0, n_chunks, step=2)
  def _main(h):
    for b in range(2):
      cc = h + b
      bi, bv, _ = bufs[b]
      wait(cc, b)

      @plsc.parallel_loop(0, CHUNKI // LANES, unroll=GUNROLL)
      def _group(g):
        sl = pl.ds(g * LANES, LANES)
        vp = bi[sl]
        vv = bv[sl]
        vx = vp & xmask
        vy = lax.shift_right_logical(vp, XSHIFT)
        for k in range(NPAIR):
          gp = plsc.load_gather(pairs[k], [vx])
          gi = plsc.bitcast(gp, jnp.int32)
          xe = plsc.bitcast(gi << 16, jnp.float32)
          xo = plsc.bitcast(gi & himask, jnp.float32)
          plsc.addupdate_scatter(accs[2 * k], [vy], xe * vv)
          plsc.addupdate_scatter(accs[2 * k + 1], [vy], xo * vv)

      @pl.when(cc + 2 < n_chunks)
      def _refill():
        issue(cc + 2, b)

  for k in range(CPS):
    pltpu.sync_copy(accs[k], out_hbm.at[cid].at[sid * CPS + k])


def _make_sc_spmm(n, n_chunks):
  mesh = plsc.VectorSubcoreMesh(core_axis_name="c", subcore_axis_name="s")
  return pl.kernel(
      _sc_body,
      out_type=jax.ShapeDtypeStruct((NC, B, n), jnp.float32),
      mesh=mesh,
      scratch_types=[pltpu.VMEM((n,), jnp.float32) for _ in range(2 + CPS)]
      + [
          pltpu.VMEM((CHUNKI,), jnp.int32),
          pltpu.VMEM((CHUNKI,), jnp.float32),
          pltpu.VMEM((CHUNKI,), jnp.int32),
          pltpu.VMEM((CHUNKI,), jnp.float32),
          pltpu.VMEM((4096,), jnp.float32),
          pltpu.VMEM((4096,), jnp.float32),
          pltpu.SemaphoreType.DMA,
          pltpu.SemaphoreType.DMA,
      ],
      compiler_params=pltpu.CompilerParams(
          use_tc_tiling_on_sc=True, needs_layout_passes=False
      ),
  )


def _combine_out(parts):
  n = parts.shape[-1]
  blk = 2048

  def body(p_ref, o_ref):
    o_ref[...] = p_ref[0] + p_ref[1]

  return pl.pallas_call(
      body,
      grid=(n // blk,),
      in_specs=[pl.BlockSpec((NC, B, blk), lambda j: (0, 0, j))],
      out_specs=pl.BlockSpec((B, blk), lambda j: (0, j)),
      out_shape=jax.ShapeDtypeStruct((B, n), jnp.float32),
  )(parts)


@jax.jit
def kernel(x, vals, idx_xs, idx_ys):
  n = x.shape[1]
  nnz = vals.shape[0]
  per_round = NC * CHUNKI
  n_chunks = -(-nnz // per_round)
  if n_chunks % 2:
    n_chunks += 1
  items = n_chunks * per_round
  pad = items - nnz

  idxp = (idx_ys << XSHIFT) | idx_xs
  idxp = jnp.concatenate([idxp, jnp.zeros((pad,), jnp.int32)])
  v = jnp.concatenate([vals, jnp.zeros((pad,), jnp.float32)])
  idxp = idxp.reshape(NC, n_chunks, CHUNKI)
  v = v.reshape(NC, n_chunks, CHUNKI)

  parts = _make_sc_spmm(n, n_chunks)(x, idxp, v)
  return _combine_out(parts)
